# baseline (speedup 1.0000x reference)
import functools

import jax
import jax.numpy as jnp
from jax.experimental import pallas as pl
from jax.experimental.pallas import tpu as pltpu

E = 8
H = 1024
C = 256
FF = 4 * H
OUT = 1024 // E
T = 2048
TOP_K = 2
TAU = 0.5

BF = 512
NF = FF // BF


def _router_kernel(ce_ref, an_ref, g_ref, w_ref, aux_ref):
    ce = ce_ref[:]
    an = an_ref[:]
    cn = ce / jnp.clip(jnp.sqrt(jnp.sum(ce * ce, axis=-1, keepdims=True)), 1e-8)
    ann = an / jnp.clip(jnp.sqrt(jnp.sum(an * an, axis=-1, keepdims=True)), 1e-8)
    logits = jax.lax.dot_general(
        cn, ann, (((1,), (1,)), ((), ())),
        precision=jax.lax.Precision.HIGHEST,
        preferred_element_type=jnp.float32) * 0.125
    x = (logits + g_ref[:]) / TAU
    x = x - jnp.max(x, axis=-1, keepdims=True)
    ex = jnp.exp(x)
    ew = ex / jnp.sum(ex, axis=-1, keepdims=True)

    e_iota = jax.lax.broadcasted_iota(jnp.int32, ew.shape, 1)
    m1 = jnp.max(ew, axis=-1, keepdims=True)
    i1 = jnp.min(jnp.where(ew == m1, e_iota, E), axis=-1, keepdims=True)
    ew_rest = jnp.where(e_iota == i1, -jnp.inf, ew)
    m2 = jnp.max(ew_rest, axis=-1, keepdims=True)
    i2 = jnp.min(jnp.where(ew_rest == m2, e_iota, E), axis=-1, keepdims=True)
    mask = (e_iota == i1) | (e_iota == i2)
    mw = jnp.where(mask, ew, 0.0)
    w_ref[:] = mw

    counts = jnp.sum(mw, axis=0, keepdims=True)
    mean = jnp.sum(counts) / E
    var = jnp.sum((counts - mean) ** 2) / (E - 1)
    std = jnp.sqrt(var)
    load = counts / jnp.sum(counts)
    ent = -jnp.sum(load * jnp.log(load + 1e-9))
    aux_ref[:] = jnp.reshape(0.5 * (std + ent), (1, 1))


def _ffn_kernel(xin_ref, w1_ref, b1_ref, w2_ref, b2_ref, wgt_ref, out_ref, acc_ref):
    e = pl.program_id(0)
    f = pl.program_id(1)

    @pl.when(f == 0)
    def _init():
        acc_ref[:] = jnp.zeros_like(acc_ref)

    xin = xin_ref[:]
    w1 = w1_ref[0]
    h1 = jax.lax.dot_general(
        xin, w1, (((1,), (1,)), ((), ())),
        precision=jax.lax.Precision.HIGHEST,
        preferred_element_type=jnp.float32)
    h1 = h1 + b1_ref[0]
    a = jax.nn.gelu(h1)
    w2 = w2_ref[0]
    acc_ref[:] += jax.lax.dot_general(
        a, w2, (((1,), (1,)), ((), ())),
        precision=jax.lax.Precision.HIGHEST,
        preferred_element_type=jnp.float32)

    @pl.when(f == NF - 1)
    def _finish():
        wg = wgt_ref[:]
        col = jax.lax.broadcasted_iota(jnp.int32, wg.shape, 1)
        wcol = jnp.sum(jnp.where(col == e, wg, 0.0), axis=1, keepdims=True)
        out_ref[:] = (acc_ref[:] + b2_ref[0]) * wcol


def kernel(h, code_emb, code_anchor, W1, b1, W2, b2):
    u = jax.random.uniform(jax.random.key(42), (T, E), minval=1e-6, maxval=1.0 - 1e-6)
    g = -jnp.log(-jnp.log(u))

    wgt, aux = pl.pallas_call(
        _router_kernel,
        out_shape=(
            jax.ShapeDtypeStruct((T, E), jnp.float32),
            jax.ShapeDtypeStruct((1, 1), jnp.float32),
        ),
    )(code_emb, code_anchor, g)

    xin = jnp.concatenate([h, code_emb], axis=-1)
    b1r = b1.reshape(E * NF, 1, BF)
    b2r = b2.reshape(E, 1, OUT)

    full = pl.pallas_call(
        _ffn_kernel,
        grid=(E, NF),
        in_specs=[
            pl.BlockSpec((T, H + C), lambda e, f: (0, 0)),
            pl.BlockSpec((1, BF, H + C), lambda e, f: (e, f, 0)),
            pl.BlockSpec((1, 1, BF), lambda e, f: (e * NF + f, 0, 0)),
            pl.BlockSpec((1, OUT, BF), lambda e, f: (e, 0, f)),
            pl.BlockSpec((1, 1, OUT), lambda e, f: (e, 0, 0)),
            pl.BlockSpec((T, E), lambda e, f: (0, 0)),
        ],
        out_specs=pl.BlockSpec((T, OUT), lambda e, f: (0, e)),
        out_shape=jax.ShapeDtypeStruct((T, E * OUT), jnp.float32),
        scratch_shapes=[pltpu.VMEM((T, OUT), jnp.float32)],
    )(xin, W1, b1r, W2, b2r, wgt)

    return full, aux[0, 0]


# speedup vs baseline: 4.4430x; 4.4430x over previous
import functools

import jax
import jax.numpy as jnp
from jax.experimental import pallas as pl
from jax.experimental.pallas import tpu as pltpu

E = 8
H = 1024
C = 256
FF = 4 * H
OUT = 1024 // E
T = 2048
TOP_K = 2
TAU = 0.5

BF = 512
NF = FF // BF


def _router_kernel(ce_ref, an_ref, g_ref, w_ref, aux_ref):
    ce = ce_ref[:]
    an = an_ref[:]
    cn = ce / jnp.clip(jnp.sqrt(jnp.sum(ce * ce, axis=-1, keepdims=True)), 1e-8)
    ann = an / jnp.clip(jnp.sqrt(jnp.sum(an * an, axis=-1, keepdims=True)), 1e-8)
    logits = jax.lax.dot_general(
        cn, ann, (((1,), (1,)), ((), ())),
        precision=jax.lax.Precision.HIGHEST,
        preferred_element_type=jnp.float32) * 0.125
    x = (logits + g_ref[:]) / TAU
    x = x - jnp.max(x, axis=-1, keepdims=True)
    ex = jnp.exp(x)
    ew = ex / jnp.sum(ex, axis=-1, keepdims=True)

    e_iota = jax.lax.broadcasted_iota(jnp.int32, ew.shape, 1)
    m1 = jnp.max(ew, axis=-1, keepdims=True)
    i1 = jnp.min(jnp.where(ew == m1, e_iota, E), axis=-1, keepdims=True)
    ew_rest = jnp.where(e_iota == i1, -jnp.inf, ew)
    m2 = jnp.max(ew_rest, axis=-1, keepdims=True)
    i2 = jnp.min(jnp.where(ew_rest == m2, e_iota, E), axis=-1, keepdims=True)
    mask = (e_iota == i1) | (e_iota == i2)
    mw = jnp.where(mask, ew, 0.0)
    w_ref[:] = mw

    counts = jnp.sum(mw, axis=0, keepdims=True)
    mean = jnp.sum(counts) / E
    var = jnp.sum((counts - mean) ** 2) / (E - 1)
    std = jnp.sqrt(var)
    load = counts / jnp.sum(counts)
    ent = -jnp.sum(load * jnp.log(load + 1e-9))
    aux_ref[:] = jnp.reshape(0.5 * (std + ent), (1, 1))


def _ffn_kernel(xin_ref, w1_ref, b1_ref, w2_ref, b2_ref, wgt_ref, out_ref, acc_ref):
    e = pl.program_id(0)
    f = pl.program_id(1)

    @pl.when(f == 0)
    def _init():
        acc_ref[:] = jnp.zeros_like(acc_ref)

    xin = xin_ref[:]
    w1 = w1_ref[0]
    h1 = jax.lax.dot_general(
        xin, w1, (((1,), (1,)), ((), ())),
        precision=jax.lax.Precision.DEFAULT,
        preferred_element_type=jnp.float32)
    h1 = h1 + b1_ref[0]
    a = jax.nn.gelu(h1)
    w2 = w2_ref[0]
    acc_ref[:] += jax.lax.dot_general(
        a, w2, (((1,), (1,)), ((), ())),
        precision=jax.lax.Precision.DEFAULT,
        preferred_element_type=jnp.float32)

    @pl.when(f == NF - 1)
    def _finish():
        wg = wgt_ref[:]
        col = jax.lax.broadcasted_iota(jnp.int32, wg.shape, 1)
        wcol = jnp.sum(jnp.where(col == e, wg, 0.0), axis=1, keepdims=True)
        out_ref[:] = (acc_ref[:] + b2_ref[0]) * wcol


def kernel(h, code_emb, code_anchor, W1, b1, W2, b2):
    u = jax.random.uniform(jax.random.key(42), (T, E), minval=1e-6, maxval=1.0 - 1e-6)
    g = -jnp.log(-jnp.log(u))

    wgt, aux = pl.pallas_call(
        _router_kernel,
        out_shape=(
            jax.ShapeDtypeStruct((T, E), jnp.float32),
            jax.ShapeDtypeStruct((1, 1), jnp.float32),
        ),
    )(code_emb, code_anchor, g)

    xin = jnp.concatenate([h, code_emb], axis=-1)
    b1r = b1.reshape(E * NF, 1, BF)
    b2r = b2.reshape(E, 1, OUT)

    full = pl.pallas_call(
        _ffn_kernel,
        grid=(E, NF),
        in_specs=[
            pl.BlockSpec((T, H + C), lambda e, f: (0, 0)),
            pl.BlockSpec((1, BF, H + C), lambda e, f: (e, f, 0)),
            pl.BlockSpec((1, 1, BF), lambda e, f: (e * NF + f, 0, 0)),
            pl.BlockSpec((1, OUT, BF), lambda e, f: (e, 0, f)),
            pl.BlockSpec((1, 1, OUT), lambda e, f: (e, 0, 0)),
            pl.BlockSpec((T, E), lambda e, f: (0, 0)),
        ],
        out_specs=pl.BlockSpec((T, OUT), lambda e, f: (0, e)),
        out_shape=jax.ShapeDtypeStruct((T, E * OUT), jnp.float32),
        scratch_shapes=[pltpu.VMEM((T, OUT), jnp.float32)],
    )(xin, W1, b1r, W2, b2r, wgt)

    return full, aux[0, 0]
